# Initial kernel scaffold; baseline (speedup 1.0000x reference)
import jax
import jax.numpy as jnp
from jax.experimental import pallas as pl


def kernel(x, post_z_mu, post_z_logD, eps, theta_raw):
    raise NotImplementedError("write your pallas kernel here")


# speedup vs baseline: 14.8118x; 14.8118x over previous
import jax
import jax.numpy as jnp
from jax.experimental import pallas as pl

_S_TILE = 512


def _body(x_ref, mu_ref, logD_ref, eps_ref, theta_ref, out_ref, loss_ref):
    b = pl.program_id(0)
    j = pl.program_id(1)

    std = jnp.exp(0.5 * logD_ref[...])
    z_b = mu_ref[...] + eps_ref[0] * std

    x_blk = x_ref[0]
    out_ref[0] = x_blk * z_b

    th = theta_ref[...]
    norm = jnp.sqrt(jnp.sum(th * th, axis=1, keepdims=True))
    tn = th / norm

    proj = jax.lax.dot_general(
        x_blk, tn, (((1,), (1,)), ((), ())), preferred_element_type=jnp.float32
    )
    zp = jax.lax.dot_general(
        z_b, tn, (((1,), (1,)), ((), ())), preferred_element_type=jnp.float32
    )
    d = proj - zp
    partial = jnp.sum(d * d, axis=(0, 1), keepdims=True)

    @pl.when((b == 0) & (j == 0))
    def _init():
        loss_ref[...] = jnp.zeros((1, 1), jnp.float32)

    loss_ref[...] += partial


def kernel(x, post_z_mu, post_z_logD, eps, theta_raw):
    B, S, D = x.shape
    P = theta_raw.shape[0]
    mu2 = post_z_mu.reshape(1, D)
    logD2 = post_z_logD.reshape(1, D)
    eps3 = eps.reshape(B, 1, D)

    out, loss = pl.pallas_call(
        _body,
        grid=(B, S // _S_TILE),
        in_specs=[
            pl.BlockSpec((1, _S_TILE, D), lambda b, j: (b, j, 0)),
            pl.BlockSpec((1, D), lambda b, j: (0, 0)),
            pl.BlockSpec((1, D), lambda b, j: (0, 0)),
            pl.BlockSpec((1, 1, D), lambda b, j: (b, 0, 0)),
            pl.BlockSpec((P, D), lambda b, j: (0, 0)),
        ],
        out_specs=[
            pl.BlockSpec((1, _S_TILE, D), lambda b, j: (b, j, 0)),
            pl.BlockSpec((1, 1), lambda b, j: (0, 0)),
        ],
        out_shape=[
            jax.ShapeDtypeStruct((B, S, D), jnp.float32),
            jax.ShapeDtypeStruct((1, 1), jnp.float32),
        ],
    )(x, mu2, logD2, eps3, theta_raw)

    w_loss = loss[0, 0] * (1.0 / (B * S * P))
    return out, w_loss


# speedup vs baseline: 17.1570x; 1.1583x over previous
import jax
import jax.numpy as jnp
from jax.experimental import pallas as pl

_S_TILE = 1024


def _body(x_ref, mu_ref, logD_ref, eps_ref, theta_ref, out_ref, loss_ref):
    b = pl.program_id(0)
    j = pl.program_id(1)

    std = jnp.exp(0.5 * logD_ref[...])
    z_b = mu_ref[...] + eps_ref[0] * std

    x_blk = x_ref[0]
    out_ref[0] = x_blk * z_b

    th = theta_ref[...]
    norm = jnp.sqrt(jnp.sum(th * th, axis=1, keepdims=True))
    tn = th / norm

    proj = jax.lax.dot_general(
        x_blk, tn, (((1,), (1,)), ((), ())), preferred_element_type=jnp.float32
    )
    zp = jax.lax.dot_general(
        z_b, tn, (((1,), (1,)), ((), ())), preferred_element_type=jnp.float32
    )
    d = proj - zp
    partial = jnp.sum(d * d, axis=(0, 1), keepdims=True)

    @pl.when((b == 0) & (j == 0))
    def _init():
        loss_ref[...] = jnp.zeros((1, 1), jnp.float32)

    loss_ref[...] += partial


def kernel(x, post_z_mu, post_z_logD, eps, theta_raw):
    B, S, D = x.shape
    P = theta_raw.shape[0]
    mu2 = post_z_mu.reshape(1, D)
    logD2 = post_z_logD.reshape(1, D)
    eps3 = eps.reshape(B, 1, D)

    out, loss = pl.pallas_call(
        _body,
        grid=(B, S // _S_TILE),
        in_specs=[
            pl.BlockSpec((1, _S_TILE, D), lambda b, j: (b, j, 0)),
            pl.BlockSpec((1, D), lambda b, j: (0, 0)),
            pl.BlockSpec((1, D), lambda b, j: (0, 0)),
            pl.BlockSpec((1, 1, D), lambda b, j: (b, 0, 0)),
            pl.BlockSpec((P, D), lambda b, j: (0, 0)),
        ],
        out_specs=[
            pl.BlockSpec((1, _S_TILE, D), lambda b, j: (b, j, 0)),
            pl.BlockSpec((1, 1), lambda b, j: (0, 0)),
        ],
        out_shape=[
            jax.ShapeDtypeStruct((B, S, D), jnp.float32),
            jax.ShapeDtypeStruct((1, 1), jnp.float32),
        ],
    )(x, mu2, logD2, eps3, theta_raw)

    w_loss = loss[0, 0] * (1.0 / (B * S * P))
    return out, w_loss


# speedup vs baseline: 17.6444x; 1.0284x over previous
import jax
import jax.numpy as jnp
from jax.experimental import pallas as pl

_S_TILE = 2048


def _body(x_ref, mu_ref, logD_ref, eps_ref, theta_ref, out_ref, loss_ref):
    b = pl.program_id(0)
    j = pl.program_id(1)

    std = jnp.exp(0.5 * logD_ref[...])
    z_b = mu_ref[...] + eps_ref[0] * std

    x_blk = x_ref[0]
    out_ref[0] = x_blk * z_b

    th = theta_ref[...]
    norm = jnp.sqrt(jnp.sum(th * th, axis=1, keepdims=True))
    tn = th / norm

    proj = jax.lax.dot_general(
        x_blk, tn, (((1,), (1,)), ((), ())), preferred_element_type=jnp.float32
    )
    zp = jax.lax.dot_general(
        z_b, tn, (((1,), (1,)), ((), ())), preferred_element_type=jnp.float32
    )
    d = proj - zp
    partial = jnp.sum(d * d, axis=(0, 1), keepdims=True)

    @pl.when((b == 0) & (j == 0))
    def _init():
        loss_ref[...] = jnp.zeros((1, 1), jnp.float32)

    loss_ref[...] += partial


def kernel(x, post_z_mu, post_z_logD, eps, theta_raw):
    B, S, D = x.shape
    P = theta_raw.shape[0]
    mu2 = post_z_mu.reshape(1, D)
    logD2 = post_z_logD.reshape(1, D)
    eps3 = eps.reshape(B, 1, D)

    out, loss = pl.pallas_call(
        _body,
        grid=(B, S // _S_TILE),
        in_specs=[
            pl.BlockSpec((1, _S_TILE, D), lambda b, j: (b, j, 0)),
            pl.BlockSpec((1, D), lambda b, j: (0, 0)),
            pl.BlockSpec((1, D), lambda b, j: (0, 0)),
            pl.BlockSpec((1, 1, D), lambda b, j: (b, 0, 0)),
            pl.BlockSpec((P, D), lambda b, j: (0, 0)),
        ],
        out_specs=[
            pl.BlockSpec((1, _S_TILE, D), lambda b, j: (b, j, 0)),
            pl.BlockSpec((1, 1), lambda b, j: (0, 0)),
        ],
        out_shape=[
            jax.ShapeDtypeStruct((B, S, D), jnp.float32),
            jax.ShapeDtypeStruct((1, 1), jnp.float32),
        ],
    )(x, mu2, logD2, eps3, theta_raw)

    w_loss = loss[0, 0] * (1.0 / (B * S * P))
    return out, w_loss
